# baseline (speedup 1.0000x reference)
import functools

import jax
import jax.numpy as jnp
from jax import lax
from jax.experimental import pallas as pl
from jax.experimental.pallas import tpu as pltpu
from jax.experimental.pallas import tpu_sc as plsc

_L = 16


@functools.lru_cache(maxsize=None)
def _make_lookup(Bt, S, V, D):
    info = plsc.get_sparse_core_info()
    NC, NS = info.num_cores, info.num_subcores
    NW = NC * NS
    assert Bt % (NW * 128) == 0 and D % 8 == 0
    BB = Bt // NW
    DT = D // 8
    SP = S + 1
    NWD = BB * SP // 2
    mesh = plsc.VectorSubcoreMesh(core_axis_name="c", subcore_axis_name="s")

    @functools.partial(
        pl.kernel,
        mesh=mesh,
        out_type=jax.ShapeDtypeStruct((S, DT, NW, 8, 128), jnp.float32),
        scratch_types=[
            pltpu.VMEM((NWD,), jnp.int32),
            pltpu.VMEM((V * D * _L,), jnp.float32),
            pltpu.VMEM((DT, 8, 128), jnp.float32),
            pltpu.VMEM((DT, 8, 128), jnp.float32),
            pltpu.SemaphoreType.DMA,
            pltpu.SemaphoreType.DMA,
        ],
        compiler_params=pltpu.CompilerParams(
            use_tc_tiling_on_sc=False, needs_layout_passes=False
        ),
    )
    def k(idx_hbm, table_hbm, out_hbm, idx_v, tab_v, pl_a, pl_b, sem_a, sem_b):
        wid = lax.axis_index("s") * NC + lax.axis_index("c")
        pltpu.sync_copy(idx_hbm.at[pl.ds(wid * NWD, NWD)], idx_v)
        pltpu.sync_copy(table_hbm, tab_v)
        lanes = lax.iota(jnp.int32, _L)
        iota_h = lanes * SP
        alt16 = (lanes & 1) << 4

        NG = BB // _L

        def compute(s, plane):
            sh = alt16 ^ ((s & 1) << 4)
            ivbs = []
            for bs in range(NG):
                pos_h = iota_h + (bs * _L * SP + s)
                w = plsc.load_gather(idx_v, [pos_h >> 1])
                iv = lax.shift_right_logical(w, sh) & 0xFFFF
                ivbs.append((iv << 9) | lanes)
            for d in range(D):
                vs = [plsc.load_gather(tab_v, [ivb | (d << 4)]) for ivb in ivbs]
                for bs in range(NG):
                    plane[d // 8, d % 8, pl.ds(bs * _L, _L)] = vs[bs]

        def fire(s, plane, sem):
            for t in range(DT):
                pltpu.async_copy(plane.at[t], out_hbm.at[s, t, wid], sem)

        def drain(plane, sem):
            for t in range(DT):
                pltpu.make_async_copy(plane.at[t], out_hbm.at[0, t, wid], sem).wait()

        compute(0, pl_a)
        fire(0, pl_a, sem_a)
        compute(1, pl_b)
        fire(1, pl_b, sem_b)

        def outer(i, c):
            s = 2 + i * 2
            drain(pl_a, sem_a)
            compute(s, pl_a)
            fire(s, pl_a, sem_a)
            drain(pl_b, sem_b)
            compute(s + 1, pl_b)
            fire(s + 1, pl_b, sem_b)
            return c

        lax.fori_loop(0, (S - 2) // 2, outer, 0)
        drain(pl_a, sem_a)
        drain(pl_b, sem_b)

    return k


def kernel(prior_info, table):
    Bt, S = prior_info.shape
    V, D = table.shape
    idx_pad = jnp.pad(prior_info.astype(jnp.int32), ((0, 0), (0, 1)))
    idx_packed = jax.lax.bitcast_convert_type(
        idx_pad.astype(jnp.uint16).reshape(-1, 2), jnp.int32
    )
    table_rep = jnp.tile(table.reshape(-1)[:, None], (1, _L)).reshape(-1)
    out5 = _make_lookup(Bt, S, V, D)(idx_packed, table_rep)
    return out5.transpose(2, 4, 0, 1, 3).reshape(Bt, S, D)


# speedup vs baseline: 2.5261x; 2.5261x over previous
import functools

import jax
import jax.numpy as jnp
from jax import lax
from jax.experimental import pallas as pl
from jax.experimental.pallas import tpu as pltpu
from jax.experimental.pallas import tpu_sc as plsc

_L = 16


@functools.lru_cache(maxsize=None)
def _make_lookup(Bt, S, V, D):
    info = plsc.get_sparse_core_info()
    NC, NS = info.num_cores, info.num_subcores
    NW = NC * NS
    assert Bt % (NW * 128) == 0 and D % 8 == 0
    BB = Bt // NW
    DT = D // 8
    DP = D + 1
    SP = S + 1
    mesh = plsc.VectorSubcoreMesh(core_axis_name="c", subcore_axis_name="s")

    @functools.partial(
        pl.kernel,
        mesh=mesh,
        out_type=jax.ShapeDtypeStruct((S, DT, NW, 8, 128), jnp.float32),
        scratch_types=[
            pltpu.VMEM((BB * SP,), jnp.int32),
            pltpu.VMEM((V * DP,), jnp.float32),
            pltpu.VMEM((DT, 8, 128), jnp.float32),
            pltpu.VMEM((DT, 8, 128), jnp.float32),
            pltpu.SemaphoreType.DMA,
            pltpu.SemaphoreType.DMA,
        ],
        compiler_params=pltpu.CompilerParams(
            use_tc_tiling_on_sc=False, needs_layout_passes=False
        ),
    )
    def k(idx_hbm, table_hbm, out_hbm, idx_v, tab_v, pl_a, pl_b, sem_a, sem_b):
        wid = lax.axis_index("s") * NC + lax.axis_index("c")
        pltpu.sync_copy(idx_hbm.at[pl.ds(wid * BB * SP, BB * SP)], idx_v)
        pltpu.sync_copy(table_hbm, tab_v)
        iota_s = lax.iota(jnp.int32, _L) * SP

        NG = BB // _L

        def compute(s, plane):
            ivds = []
            for bs in range(NG):
                pos = iota_s + (bs * _L * SP + s)
                iv = plsc.load_gather(idx_v, [pos])
                ivds.append(iv * DP)
            for d in range(D):
                vs = [plsc.load_gather(tab_v, [ivd + d]) for ivd in ivds]
                for bs in range(NG):
                    plane[d // 8, d % 8, pl.ds(bs * _L, _L)] = vs[bs]

        def fire(s, plane, sem):
            for t in range(DT):
                pltpu.async_copy(plane.at[t], out_hbm.at[s, t, wid], sem)

        def drain(plane, sem):
            for t in range(DT):
                pltpu.make_async_copy(plane.at[t], out_hbm.at[0, t, wid], sem).wait()

        compute(0, pl_a)
        fire(0, pl_a, sem_a)
        compute(1, pl_b)
        fire(1, pl_b, sem_b)

        def outer(i, c):
            s = 2 + i * 2
            drain(pl_a, sem_a)
            compute(s, pl_a)
            fire(s, pl_a, sem_a)
            drain(pl_b, sem_b)
            compute(s + 1, pl_b)
            fire(s + 1, pl_b, sem_b)
            return c

        lax.fori_loop(0, (S - 2) // 2, outer, 0)
        drain(pl_a, sem_a)
        drain(pl_b, sem_b)

    return k


def kernel(prior_info, table):
    Bt, S = prior_info.shape
    V, D = table.shape
    idx = jnp.pad(prior_info.astype(jnp.int32), ((0, 0), (0, 1))).reshape(-1)
    table_padded = jnp.pad(table, ((0, 0), (0, 1))).reshape(-1)
    out5 = _make_lookup(Bt, S, V, D)(idx, table_padded)
    return out5.transpose(2, 4, 0, 1, 3).reshape(Bt, S, D)
